# baseline (speedup 1.0000x reference)
import numpy as np
import jax
import jax.numpy as jnp
from jax.experimental import pallas as pl
from jax.experimental.pallas import tpu as pltpu

K = 16
QB = 128
CB = 512
BIG = 2**30

_REVBITS = int(np.log2(CB))
_BITREV = np.array(
    [int(format(p, f"0{_REVBITS}b")[::-1], 2) for p in range(CB)],
    dtype=np.int32,
)


def _lexmin_tree(v, i):
    r = v.shape[0]
    while r > 1:
        h = r // 2
        cmp = v[:h] <= v[h:]
        v = jnp.where(cmp, v[:h], v[h:])
        i = jnp.where(cmp, i[:h], i[h:])
        r = h
    return v, i


def _extract_topk_tree(d, io, nk):
    kiota = jax.lax.broadcasted_iota(jnp.int32, (nk, d.shape[1]), 0)
    cv = jnp.zeros((nk, d.shape[1]), jnp.float32)
    cg = jnp.zeros((nk, d.shape[1]), jnp.int32)
    for k in range(nk):
        m, gi = _lexmin_tree(d, io)
        cv = jnp.where(kiota == k, m, cv)
        cg = jnp.where(kiota == k, gi, cg)
        if k < nk - 1:
            d = jnp.where(io == gi, jnp.float32(jnp.inf), d)
    return cv, cg


def _extract_topk_flat(d, io, nk):
    kiota = jax.lax.broadcasted_iota(jnp.int32, (nk, d.shape[1]), 0)
    cv = jnp.zeros((nk, d.shape[1]), jnp.float32)
    cg = jnp.zeros((nk, d.shape[1]), jnp.int32)
    for k in range(nk):
        m = jnp.min(d, axis=0, keepdims=True)
        gi = jnp.min(jnp.where(d == m, io, BIG), axis=0, keepdims=True)
        cv = jnp.where(kiota == k, m, cv)
        cg = jnp.where(kiota == k, gi, cg)
        if k < nk - 1:
            d = jnp.where(io == gi, jnp.float32(jnp.inf), d)
    return cv, cg


def _knn_kernel(q_ref, s_ref, tbl_ref, out_ref, bv_ref, bg_ref):
    c = pl.program_id(2)

    @pl.when(c == 0)
    def init():
        bv_ref[...] = jnp.full(bv_ref.shape, jnp.inf, jnp.float32)
        bg_ref[...] = jnp.full(bg_ref.shape, BIG, jnp.int32)

    q = q_ref[0]
    s = s_ref[0]
    sn = jnp.sum(s * s, axis=1, keepdims=True)
    qn = jnp.sum(q * q, axis=1)
    sq = jax.lax.dot_general(
        s, q, (((1,), (1,)), ((), ())),
        precision=jax.lax.Precision.HIGHEST,
        preferred_element_type=jnp.float32,
    )
    d = (sn - 2.0 * sq) + qn[None, :]
    io = tbl_ref[...] + c * CB
    cv, cg = _extract_topk_tree(d, io, K)

    vcomb = jnp.concatenate([bv_ref[...], cv], axis=0)
    gcomb = jnp.concatenate([bg_ref[...], cg], axis=0)
    bv, bg = _extract_topk_flat(vcomb, gcomb, K)
    bv_ref[...] = bv
    bg_ref[...] = bg

    @pl.when(c == pl.num_programs(2) - 1)
    def write_out():
        out_ref[0] = bg.T


def kernel(query_points, sample_points):
    B, Q, D = query_points.shape
    _, N, _ = sample_points.shape
    s_perm = (
        sample_points.reshape((B, N // CB) + (2,) * _REVBITS + (D,))
        .transpose((0, 1) + tuple(range(1 + _REVBITS, 1, -1)) + (2 + _REVBITS,))
        .reshape(B, N, D)
    )
    tbl = jnp.asarray(np.tile(_BITREV[:, None], (1, QB)))
    grid = (B, Q // QB, N // CB)
    out = pl.pallas_call(
        _knn_kernel,
        grid=grid,
        in_specs=[
            pl.BlockSpec((1, QB, D), lambda b, i, c: (b, i, 0)),
            pl.BlockSpec((1, CB, D), lambda b, i, c: (b, c, 0)),
            pl.BlockSpec((CB, QB), lambda b, i, c: (0, 0)),
        ],
        out_specs=pl.BlockSpec((1, QB, K), lambda b, i, c: (b, i, 0)),
        out_shape=jax.ShapeDtypeStruct((B, Q, K), jnp.int32),
        scratch_shapes=[
            pltpu.VMEM((K, QB), jnp.float32),
            pltpu.VMEM((K, QB), jnp.int32),
        ],
    )(query_points, s_perm, tbl)
    return out


# speedup vs baseline: 1.0305x; 1.0305x over previous
import numpy as np
import jax
import jax.numpy as jnp
from jax.experimental import pallas as pl
from jax.experimental.pallas import tpu as pltpu

K = 16
QB = 128
CB = 512
BIG = 2**30

RB = 256
_REVBITS = int(np.log2(RB))
_BITREV = np.array(
    [int(format(p, f"0{_REVBITS}b")[::-1], 2) for p in range(RB)],
    dtype=np.int32,
)


def _lexmin_tree(v, i):
    r = v.shape[0]
    while r > 1:
        h = r // 2
        cmp = v[:h] <= v[h:]
        v = jnp.where(cmp, v[:h], v[h:])
        i = jnp.where(cmp, i[:h], i[h:])
        r = h
    return v, i


def _extract_topk_tree(d, io, nk):
    kiota = jax.lax.broadcasted_iota(jnp.int32, (nk, d.shape[1]), 0)
    cv = jnp.zeros((nk, d.shape[1]), jnp.float32)
    cg = jnp.zeros((nk, d.shape[1]), jnp.int32)
    for k in range(nk):
        m, gi = _lexmin_tree(d, io)
        cv = jnp.where(kiota == k, m, cv)
        cg = jnp.where(kiota == k, gi, cg)
        if k < nk - 1:
            d = jnp.where(io == gi, jnp.float32(jnp.inf), d)
    return cv, cg


def _extract_topk_flat(d, io, nk):
    kiota = jax.lax.broadcasted_iota(jnp.int32, (nk, d.shape[1]), 0)
    cv = jnp.zeros((nk, d.shape[1]), jnp.float32)
    cg = jnp.zeros((nk, d.shape[1]), jnp.int32)
    for k in range(nk):
        m = jnp.min(d, axis=0, keepdims=True)
        gi = jnp.min(jnp.where(d == m, io, BIG), axis=0, keepdims=True)
        cv = jnp.where(kiota == k, m, cv)
        cg = jnp.where(kiota == k, gi, cg)
        if k < nk - 1:
            d = jnp.where(io == gi, jnp.float32(jnp.inf), d)
    return cv, cg


def _knn_kernel(q_ref, s_ref, tbl_ref, out_ref, bv_ref, bg_ref):
    c = pl.program_id(2)

    @pl.when(c == 0)
    def init():
        bv_ref[...] = jnp.full(bv_ref.shape, jnp.inf, jnp.float32)
        bg_ref[...] = jnp.full(bg_ref.shape, BIG, jnp.int32)

    q = q_ref[0]
    s = s_ref[0]
    sn = jnp.sum(s * s, axis=1, keepdims=True)
    qn = jnp.sum(q * q, axis=1)
    sq = jax.lax.dot_general(
        s, q, (((1,), (1,)), ((), ())),
        precision=jax.lax.Precision.HIGHEST,
        preferred_element_type=jnp.float32,
    )
    d = (sn - 2.0 * sq) + qn[None, :]
    io = tbl_ref[...] + c * CB
    cv0, cg0 = _extract_topk_tree(d[:RB], io[:RB], K)
    cv1, cg1 = _extract_topk_tree(d[RB:], io[RB:], K)

    vcomb = jnp.concatenate([bv_ref[...], cv0, cv1], axis=0)
    gcomb = jnp.concatenate([bg_ref[...], cg0, cg1], axis=0)
    bv, bg = _extract_topk_flat(vcomb, gcomb, K)
    bv_ref[...] = bv
    bg_ref[...] = bg
    out_ref[0] = bg


def kernel(query_points, sample_points):
    B, Q, D = query_points.shape
    _, N, _ = sample_points.shape
    s_perm = (
        sample_points.reshape((B, N // RB) + (2,) * _REVBITS + (D,))
        .transpose((0, 1) + tuple(range(1 + _REVBITS, 1, -1)) + (2 + _REVBITS,))
        .reshape(B, N, D)
    )
    tbl_col = np.concatenate(
        [_BITREV + j * RB for j in range(CB // RB)]
    ).astype(np.int32)
    tbl = jnp.asarray(np.tile(tbl_col[:, None], (1, QB)))
    grid = (B, Q // QB, N // CB)
    out = pl.pallas_call(
        _knn_kernel,
        grid=grid,
        in_specs=[
            pl.BlockSpec((1, QB, D), lambda b, i, c: (b, i, 0)),
            pl.BlockSpec((1, CB, D), lambda b, i, c: (b, c, 0)),
            pl.BlockSpec((CB, QB), lambda b, i, c: (0, 0)),
        ],
        out_specs=pl.BlockSpec((1, K, QB), lambda b, i, c: (b, 0, i)),
        out_shape=jax.ShapeDtypeStruct((B, K, Q), jnp.int32),
        scratch_shapes=[
            pltpu.VMEM((K, QB), jnp.float32),
            pltpu.VMEM((K, QB), jnp.int32),
        ],
    )(query_points, s_perm, tbl)
    return jnp.transpose(out, (0, 2, 1))


# speedup vs baseline: 1.2181x; 1.1821x over previous
import numpy as np
import jax
import jax.numpy as jnp
from jax.experimental import pallas as pl
from jax.experimental.pallas import tpu as pltpu

K = 16
QB = 128
CB = 1024
BIG = 2**30

RB = 256
_REVBITS = int(np.log2(RB))
_BITREV = np.array(
    [int(format(p, f"0{_REVBITS}b")[::-1], 2) for p in range(RB)],
    dtype=np.int32,
)


def _lexmin_tree(v, i):
    r = v.shape[0]
    while r > 1:
        h = r // 2
        cmp = v[:h] <= v[h:]
        v = jnp.where(cmp, v[:h], v[h:])
        i = jnp.where(cmp, i[:h], i[h:])
        r = h
    return v, i


def _extract_topk_tree(d, io, nk):
    kiota = jax.lax.broadcasted_iota(jnp.int32, (nk, d.shape[1]), 0)
    cv = jnp.zeros((nk, d.shape[1]), jnp.float32)
    cg = jnp.zeros((nk, d.shape[1]), jnp.int32)
    for k in range(nk):
        m, gi = _lexmin_tree(d, io)
        cv = jnp.where(kiota == k, m, cv)
        cg = jnp.where(kiota == k, gi, cg)
        if k < nk - 1:
            d = jnp.where(io == gi, jnp.float32(jnp.inf), d)
    return cv, cg


def _extract_topk_flat(d, io, nk):
    kiota = jax.lax.broadcasted_iota(jnp.int32, (nk, d.shape[1]), 0)
    cv = jnp.zeros((nk, d.shape[1]), jnp.float32)
    cg = jnp.zeros((nk, d.shape[1]), jnp.int32)
    for k in range(nk):
        m = jnp.min(d, axis=0, keepdims=True)
        gi = jnp.min(jnp.where(d == m, io, BIG), axis=0, keepdims=True)
        cv = jnp.where(kiota == k, m, cv)
        cg = jnp.where(kiota == k, gi, cg)
        if k < nk - 1:
            d = jnp.where(io == gi, jnp.float32(jnp.inf), d)
    return cv, cg


def _knn_kernel(q_ref, s_ref, tbl_ref, out_ref, bv_ref, bg_ref):
    c = pl.program_id(2)

    @pl.when(c == 0)
    def init():
        bv_ref[...] = jnp.full(bv_ref.shape, jnp.inf, jnp.float32)
        bg_ref[...] = jnp.full(bg_ref.shape, BIG, jnp.int32)

    q = q_ref[0]
    s = s_ref[0]
    sn = jnp.sum(s * s, axis=1, keepdims=True)
    qn = jnp.sum(q * q, axis=1)
    sq = jax.lax.dot_general(
        s, q, (((1,), (1,)), ((), ())),
        precision=jax.lax.Precision.HIGHEST,
        preferred_element_type=jnp.float32,
    )
    d = (sn - 2.0 * sq) + qn[None, :]
    io = tbl_ref[...] + c * CB
    segs = [
        _extract_topk_tree(d[j * RB:(j + 1) * RB], io[j * RB:(j + 1) * RB], K)
        for j in range(CB // RB)
    ]

    vcomb = jnp.concatenate([bv_ref[...]] + [s[0] for s in segs], axis=0)
    gcomb = jnp.concatenate([bg_ref[...]] + [s[1] for s in segs], axis=0)
    bv, bg = _extract_topk_flat(vcomb, gcomb, K)
    bv_ref[...] = bv
    bg_ref[...] = bg
    out_ref[0] = bg


def kernel(query_points, sample_points):
    B, Q, D = query_points.shape
    _, N, _ = sample_points.shape
    s_perm = (
        sample_points.reshape((B, N // RB) + (2,) * _REVBITS + (D,))
        .transpose((0, 1) + tuple(range(1 + _REVBITS, 1, -1)) + (2 + _REVBITS,))
        .reshape(B, N, D)
    )
    tbl_col = np.concatenate(
        [_BITREV + j * RB for j in range(CB // RB)]
    ).astype(np.int32)
    tbl = jnp.asarray(np.tile(tbl_col[:, None], (1, QB)))
    grid = (B, Q // QB, N // CB)
    out = pl.pallas_call(
        _knn_kernel,
        grid=grid,
        in_specs=[
            pl.BlockSpec((1, QB, D), lambda b, i, c: (b, i, 0)),
            pl.BlockSpec((1, CB, D), lambda b, i, c: (b, c, 0)),
            pl.BlockSpec((CB, QB), lambda b, i, c: (0, 0)),
        ],
        out_specs=pl.BlockSpec((1, K, QB), lambda b, i, c: (b, 0, i)),
        out_shape=jax.ShapeDtypeStruct((B, K, Q), jnp.int32),
        scratch_shapes=[
            pltpu.VMEM((K, QB), jnp.float32),
            pltpu.VMEM((K, QB), jnp.int32),
        ],
    )(query_points, s_perm, tbl)
    return jnp.transpose(out, (0, 2, 1))


# speedup vs baseline: 1.3368x; 1.0975x over previous
import numpy as np
import jax
import jax.numpy as jnp
from jax.experimental import pallas as pl
from jax.experimental.pallas import tpu as pltpu

K = 16
QB = 128
CB = 2048
BIG = 2**30

RB = 256
_REVBITS = int(np.log2(RB))
_BITREV = np.array(
    [int(format(p, f"0{_REVBITS}b")[::-1], 2) for p in range(RB)],
    dtype=np.int32,
)


def _lexmin_tree(v, i):
    r = v.shape[0]
    while r > 1:
        h = r // 2
        cmp = v[:h] <= v[h:]
        v = jnp.where(cmp, v[:h], v[h:])
        i = jnp.where(cmp, i[:h], i[h:])
        r = h
    return v, i


def _extract_topk_tree(d, io, nk):
    kiota = jax.lax.broadcasted_iota(jnp.int32, (nk, d.shape[1]), 0)
    cv = jnp.zeros((nk, d.shape[1]), jnp.float32)
    cg = jnp.zeros((nk, d.shape[1]), jnp.int32)
    for k in range(nk):
        m, gi = _lexmin_tree(d, io)
        cv = jnp.where(kiota == k, m, cv)
        cg = jnp.where(kiota == k, gi, cg)
        if k < nk - 1:
            d = jnp.where(io == gi, jnp.float32(jnp.inf), d)
    return cv, cg


def _extract_topk_flat(d, io, nk):
    kiota = jax.lax.broadcasted_iota(jnp.int32, (nk, d.shape[1]), 0)
    cv = jnp.zeros((nk, d.shape[1]), jnp.float32)
    cg = jnp.zeros((nk, d.shape[1]), jnp.int32)
    for k in range(nk):
        m = jnp.min(d, axis=0, keepdims=True)
        gi = jnp.min(jnp.where(d == m, io, BIG), axis=0, keepdims=True)
        cv = jnp.where(kiota == k, m, cv)
        cg = jnp.where(kiota == k, gi, cg)
        if k < nk - 1:
            d = jnp.where(io == gi, jnp.float32(jnp.inf), d)
    return cv, cg


def _knn_kernel(q_ref, s_ref, tbl_ref, out_ref, bv_ref, bg_ref):
    c = pl.program_id(2)

    @pl.when(c == 0)
    def init():
        bv_ref[...] = jnp.full(bv_ref.shape, jnp.inf, jnp.float32)
        bg_ref[...] = jnp.full(bg_ref.shape, BIG, jnp.int32)

    q = q_ref[0]
    s = s_ref[0]
    sn = jnp.sum(s * s, axis=1, keepdims=True)
    qn = jnp.sum(q * q, axis=1)
    sq = jax.lax.dot_general(
        s, q, (((1,), (1,)), ((), ())),
        precision=jax.lax.Precision.HIGHEST,
        preferred_element_type=jnp.float32,
    )
    d = (sn - 2.0 * sq) + qn[None, :]
    io = tbl_ref[...] + c * CB
    segs = [
        _extract_topk_tree(d[j * RB:(j + 1) * RB], io[j * RB:(j + 1) * RB], K)
        for j in range(CB // RB)
    ]

    vcomb = jnp.concatenate([bv_ref[...]] + [s[0] for s in segs], axis=0)
    gcomb = jnp.concatenate([bg_ref[...]] + [s[1] for s in segs], axis=0)
    bv, bg = _extract_topk_flat(vcomb, gcomb, K)
    bv_ref[...] = bv
    bg_ref[...] = bg
    out_ref[0] = bg


def kernel(query_points, sample_points):
    B, Q, D = query_points.shape
    _, N, _ = sample_points.shape
    s_perm = (
        sample_points.reshape((B, N // RB) + (2,) * _REVBITS + (D,))
        .transpose((0, 1) + tuple(range(1 + _REVBITS, 1, -1)) + (2 + _REVBITS,))
        .reshape(B, N, D)
    )
    tbl_col = np.concatenate(
        [_BITREV + j * RB for j in range(CB // RB)]
    ).astype(np.int32)
    tbl = jnp.asarray(np.tile(tbl_col[:, None], (1, QB)))
    grid = (B, Q // QB, N // CB)
    out = pl.pallas_call(
        _knn_kernel,
        grid=grid,
        in_specs=[
            pl.BlockSpec((1, QB, D), lambda b, i, c: (b, i, 0)),
            pl.BlockSpec((1, CB, D), lambda b, i, c: (b, c, 0)),
            pl.BlockSpec((CB, QB), lambda b, i, c: (0, 0)),
        ],
        out_specs=pl.BlockSpec((1, K, QB), lambda b, i, c: (b, 0, i)),
        out_shape=jax.ShapeDtypeStruct((B, K, Q), jnp.int32),
        scratch_shapes=[
            pltpu.VMEM((K, QB), jnp.float32),
            pltpu.VMEM((K, QB), jnp.int32),
        ],
    )(query_points, s_perm, tbl)
    return jnp.transpose(out, (0, 2, 1))


# speedup vs baseline: 1.4238x; 1.0651x over previous
import numpy as np
import jax
import jax.numpy as jnp
from jax.experimental import pallas as pl
from jax.experimental.pallas import tpu as pltpu

K = 16
QB = 128
CB = 8192
BIG = 2**30

RB = 256
_REVBITS = int(np.log2(RB))
_BITREV = np.array(
    [int(format(p, f"0{_REVBITS}b")[::-1], 2) for p in range(RB)],
    dtype=np.int32,
)


def _lexmin_tree(v, i):
    r = v.shape[0]
    while r > 1:
        h = r // 2
        cmp = v[:h] <= v[h:]
        v = jnp.where(cmp, v[:h], v[h:])
        i = jnp.where(cmp, i[:h], i[h:])
        r = h
    return v, i


def _extract_topk_tree(d, io, nk):
    kiota = jax.lax.broadcasted_iota(jnp.int32, (nk, d.shape[1]), 0)
    cv = jnp.zeros((nk, d.shape[1]), jnp.float32)
    cg = jnp.zeros((nk, d.shape[1]), jnp.int32)
    for k in range(nk):
        m, gi = _lexmin_tree(d, io)
        cv = jnp.where(kiota == k, m, cv)
        cg = jnp.where(kiota == k, gi, cg)
        if k < nk - 1:
            d = jnp.where(io == gi, jnp.float32(jnp.inf), d)
    return cv, cg


def _extract_topk_flat(d, io, nk):
    kiota = jax.lax.broadcasted_iota(jnp.int32, (nk, d.shape[1]), 0)
    cv = jnp.zeros((nk, d.shape[1]), jnp.float32)
    cg = jnp.zeros((nk, d.shape[1]), jnp.int32)
    for k in range(nk):
        m = jnp.min(d, axis=0, keepdims=True)
        gi = jnp.min(jnp.where(d == m, io, BIG), axis=0, keepdims=True)
        cv = jnp.where(kiota == k, m, cv)
        cg = jnp.where(kiota == k, gi, cg)
        if k < nk - 1:
            d = jnp.where(io == gi, jnp.float32(jnp.inf), d)
    return cv, cg


def _knn_kernel(q_ref, s_ref, tbl_ref, out_ref, bv_ref, bg_ref):
    c = pl.program_id(2)

    @pl.when(c == 0)
    def init():
        bv_ref[...] = jnp.full(bv_ref.shape, jnp.inf, jnp.float32)
        bg_ref[...] = jnp.full(bg_ref.shape, BIG, jnp.int32)

    q = q_ref[0]
    s = s_ref[0]
    sn = jnp.sum(s * s, axis=1, keepdims=True)
    qn = jnp.sum(q * q, axis=1)
    sq = jax.lax.dot_general(
        s, q, (((1,), (1,)), ((), ())),
        precision=jax.lax.Precision.HIGHEST,
        preferred_element_type=jnp.float32,
    )
    d = (sn - 2.0 * sq) + qn[None, :]
    io = tbl_ref[...] + c * CB
    segs = [
        _extract_topk_tree(d[j * RB:(j + 1) * RB], io[j * RB:(j + 1) * RB], K)
        for j in range(CB // RB)
    ]

    vcomb = jnp.concatenate([bv_ref[...]] + [s[0] for s in segs], axis=0)
    gcomb = jnp.concatenate([bg_ref[...]] + [s[1] for s in segs], axis=0)
    bv, bg = _extract_topk_flat(vcomb, gcomb, K)
    bv_ref[...] = bv
    bg_ref[...] = bg
    out_ref[0] = bg


def kernel(query_points, sample_points):
    B, Q, D = query_points.shape
    _, N, _ = sample_points.shape
    s_perm = (
        sample_points.reshape((B, N // RB) + (2,) * _REVBITS + (D,))
        .transpose((0, 1) + tuple(range(1 + _REVBITS, 1, -1)) + (2 + _REVBITS,))
        .reshape(B, N, D)
    )
    tbl_col = np.concatenate(
        [_BITREV + j * RB for j in range(CB // RB)]
    ).astype(np.int32)
    tbl = jnp.asarray(np.tile(tbl_col[:, None], (1, QB)))
    grid = (B, Q // QB, N // CB)
    out = pl.pallas_call(
        _knn_kernel,
        grid=grid,
        in_specs=[
            pl.BlockSpec((1, QB, D), lambda b, i, c: (b, i, 0)),
            pl.BlockSpec((1, CB, D), lambda b, i, c: (b, c, 0)),
            pl.BlockSpec((CB, QB), lambda b, i, c: (0, 0)),
        ],
        out_specs=pl.BlockSpec((1, K, QB), lambda b, i, c: (b, 0, i)),
        out_shape=jax.ShapeDtypeStruct((B, K, Q), jnp.int32),
        scratch_shapes=[
            pltpu.VMEM((K, QB), jnp.float32),
            pltpu.VMEM((K, QB), jnp.int32),
        ],
    )(query_points, s_perm, tbl)
    return jnp.transpose(out, (0, 2, 1))
